# Initial kernel scaffold; baseline (speedup 1.0000x reference)
import jax
import jax.numpy as jnp
from jax.experimental import pallas as pl


def kernel(x, edge_index, W1, b1, W2, b2, W3, b3):
    raise NotImplementedError("write your pallas kernel here")


# speedup vs baseline: 7.0618x; 7.0618x over previous
import functools

import jax
import jax.numpy as jnp
from jax import lax
from jax.experimental import pallas as pl
from jax.experimental.pallas import tpu as pltpu
from jax.experimental.pallas import tpu_sc as plsc

N = 10000
D = 128
LANE = 128
NC, NS = 2, 16
NT = NC * NS
N_ACC = 10240
RPT = N_ACC // NS
DEGW = 128

_sc_mesh = plsc.VectorSubcoreMesh(core_axis_name="c", subcore_axis_name="s")


def _deg_body(dst_hbm, zeros_hbm, ones_hbm, out_hbm, dst_v, ones_v, acc, *, ch):
    c = lax.axis_index("c")
    s = lax.axis_index("s")
    wid = c * NS + s
    pltpu.sync_copy(dst_hbm.at[pl.ds(wid * ch, ch)], dst_v)
    pltpu.sync_copy(ones_hbm, ones_v)
    pltpu.sync_copy(zeros_hbm, acc.at[pl.ds(s * RPT, RPT)])
    plsc.subcore_barrier()

    def body(j, carry):
        pltpu.sync_copy(ones_v, acc.at[dst_v.at[j]], add=True)
        return carry

    lax.fori_loop(0, ch, body, 0)
    plsc.subcore_barrier()
    pltpu.sync_copy(acc.at[pl.ds(s * RPT, RPT)], out_hbm.at[c, pl.ds(s * RPT, RPT)])


def _agg_body(hs_hbm, src_hbm, dst_hbm, zeros_hbm, out_hbm, src_v, dst_v, rows_v, acc, *, ch):
    c = lax.axis_index("c")
    s = lax.axis_index("s")
    wid = c * NS + s
    pltpu.sync_copy(src_hbm.at[pl.ds(wid * ch, ch)], src_v)
    pltpu.sync_copy(dst_hbm.at[pl.ds(wid * ch, ch)], dst_v)
    pltpu.sync_copy(zeros_hbm, acc.at[pl.ds(s * RPT, RPT)])
    plsc.subcore_barrier()

    def body(j, carry):
        pltpu.sync_copy(hs_hbm.at[src_v.at[j]], rows_v)
        pltpu.sync_copy(rows_v, acc.at[dst_v.at[j]], add=True)
        return carry

    lax.fori_loop(0, ch, body, 0)
    plsc.subcore_barrier()
    pltpu.sync_copy(acc.at[pl.ds(s * RPT, RPT)], out_hbm.at[c, pl.ds(s * RPT, RPT)])


def _make_deg(ch):
    return pl.kernel(
        functools.partial(_deg_body, ch=ch),
        out_type=jax.ShapeDtypeStruct((NC, N_ACC, DEGW), jnp.float32),
        mesh=_sc_mesh,
        scratch_types=[
            pltpu.VMEM((ch, LANE), jnp.int32),
            pltpu.VMEM((LANE, DEGW), jnp.float32),
            pltpu.VMEM_SHARED((N_ACC, DEGW), jnp.float32),
        ],
    )


def _make_agg(ch):
    return pl.kernel(
        functools.partial(_agg_body, ch=ch),
        out_type=jax.ShapeDtypeStruct((NC, N_ACC, D), jnp.float32),
        mesh=_sc_mesh,
        scratch_types=[
            pltpu.VMEM((ch, LANE), jnp.int32),
            pltpu.VMEM((ch, LANE), jnp.int32),
            pltpu.VMEM((LANE, D), jnp.float32),
            pltpu.VMEM_SHARED((N_ACC, D), jnp.float32),
        ],
    )


_RB = 400
_GRID = N // _RB


def _tc_first_body(deg_ref, x_ref, w_ref, dis_ref, hs_ref):
    deg = deg_ref[0, :, 0:1] + deg_ref[1, :, 0:1] + 1.0
    dis = lax.rsqrt(jnp.maximum(deg, 1.0))
    dis_ref[...] = dis
    hs_ref[...] = dis * jnp.dot(x_ref[...], w_ref[...], preferred_element_type=jnp.float32)


def _tc_mid_body(agg_ref, hs_ref, dis_ref, b_ref, w_ref, out_ref):
    dis = dis_ref[...]
    t = (agg_ref[0] + agg_ref[1] + hs_ref[...]) * dis + b_ref[...]
    h = jnp.maximum(t, 0.0)
    out_ref[...] = dis * jnp.dot(h, w_ref[...], preferred_element_type=jnp.float32)


def _tc_last_body(agg_ref, hs_ref, dis_ref, b_ref, out_ref):
    out_ref[...] = (agg_ref[0] + agg_ref[1] + hs_ref[...]) * dis_ref[...] + b_ref[...]


_tc_first = pl.pallas_call(
    _tc_first_body,
    grid=(_GRID,),
    in_specs=[
        pl.BlockSpec((NC, _RB, DEGW), lambda i: (0, i, 0)),
        pl.BlockSpec((_RB, D), lambda i: (i, 0)),
        pl.BlockSpec((D, D), lambda i: (0, 0)),
    ],
    out_specs=[
        pl.BlockSpec((_RB, 1), lambda i: (i, 0)),
        pl.BlockSpec((_RB, D), lambda i: (i, 0)),
    ],
    out_shape=[
        jax.ShapeDtypeStruct((N, 1), jnp.float32),
        jax.ShapeDtypeStruct((N, D), jnp.float32),
    ],
)

_tc_mid = pl.pallas_call(
    _tc_mid_body,
    grid=(_GRID,),
    in_specs=[
        pl.BlockSpec((NC, _RB, D), lambda i: (0, i, 0)),
        pl.BlockSpec((_RB, D), lambda i: (i, 0)),
        pl.BlockSpec((_RB, 1), lambda i: (i, 0)),
        pl.BlockSpec((1, D), lambda i: (0, 0)),
        pl.BlockSpec((D, D), lambda i: (0, 0)),
    ],
    out_specs=pl.BlockSpec((_RB, D), lambda i: (i, 0)),
    out_shape=jax.ShapeDtypeStruct((N, D), jnp.float32),
)

_tc_last = pl.pallas_call(
    _tc_last_body,
    grid=(_GRID,),
    in_specs=[
        pl.BlockSpec((NC, _RB, D), lambda i: (0, i, 0)),
        pl.BlockSpec((_RB, D), lambda i: (i, 0)),
        pl.BlockSpec((_RB, 1), lambda i: (i, 0)),
        pl.BlockSpec((1, D), lambda i: (0, 0)),
    ],
    out_specs=pl.BlockSpec((_RB, D), lambda i: (i, 0)),
    out_shape=jax.ShapeDtypeStruct((N, D), jnp.float32),
)


def kernel(x, edge_index, W1, b1, W2, b2, W3, b3):
    n, d = x.shape
    e = edge_index.shape[1]
    ch = -(-e // (NT * LANE))
    ch = -(-ch // 8) * 8
    e_pad = NT * ch * LANE
    pad = e_pad - e

    src = jnp.concatenate([edge_index[0], jnp.zeros((pad,), jnp.int32)])
    dst = jnp.concatenate([edge_index[1], jnp.full((pad,), n, jnp.int32)])
    src2d = src.reshape(NT * ch, LANE)
    dst2d = dst.reshape(NT * ch, LANE)

    zrows = jnp.zeros((RPT, D), jnp.float32)
    ones = jnp.ones((LANE, DEGW), jnp.float32)

    deg_p = _make_deg(ch)(dst2d, zrows, ones)
    agg_fn = _make_agg(ch)

    dis, hs = _tc_first(deg_p, x, W1)

    agg = agg_fn(hs, src2d, dst2d, zrows)
    hs = _tc_mid(agg, hs, dis, b1.reshape(1, D), W2)

    agg = agg_fn(hs, src2d, dst2d, zrows)
    hs = _tc_mid(agg, hs, dis, b2.reshape(1, D), W3)

    agg = agg_fn(hs, src2d, dst2d, zrows)
    out = _tc_last(agg, hs, dis, b3.reshape(1, D))
    return out


# speedup vs baseline: 7.9108x; 1.1202x over previous
import functools

import jax
import jax.numpy as jnp
from jax import lax
from jax.experimental import pallas as pl
from jax.experimental.pallas import tpu as pltpu
from jax.experimental.pallas import tpu_sc as plsc

N = 10000
D = 128
LANE = 128
NC, NS = 2, 16
NT = NC * NS
N_ACC = 10240
RPT = N_ACC // NS
DEGW = 128

_sc_mesh = plsc.VectorSubcoreMesh(core_axis_name="c", subcore_axis_name="s")


def _deg_body(dst_hbm, zeros_hbm, ones_hbm, out_hbm, dst_v, ones_v, sem, acc, *, ch):
    c = lax.axis_index("c")
    s = lax.axis_index("s")
    wid = c * NS + s
    pltpu.sync_copy(dst_hbm.at[pl.ds(wid * ch, ch)], dst_v)
    pltpu.sync_copy(ones_hbm, ones_v)
    pltpu.sync_copy(zeros_hbm, acc.at[pl.ds(s * RPT, RPT)])
    plsc.subcore_barrier()

    def fire(j, carry):
        pltpu.async_copy(ones_v, acc.at[dst_v.at[j]], sem, add=True)
        return carry

    lax.fori_loop(0, ch, fire, 0)

    def drain(j, carry):
        pltpu.make_async_copy(ones_v, acc.at[dst_v.at[j]], sem).wait()
        return carry

    lax.fori_loop(0, ch, drain, 0)
    plsc.subcore_barrier()
    pltpu.sync_copy(acc.at[pl.ds(s * RPT, RPT)], out_hbm.at[c, pl.ds(s * RPT, RPT)])


NBUF = 2


def _agg_body(hs_hbm, src_hbm, dst_hbm, zeros_hbm, out_hbm, src_v, dstr_v,
              rows_v, gsems, dsems, ssems, acc, *, ch):
    c = lax.axis_index("c")
    s = lax.axis_index("s")
    wid = c * NS + s
    pltpu.sync_copy(src_hbm.at[pl.ds(wid * ch, ch)], src_v)
    pltpu.sync_copy(zeros_hbm, acc.at[pl.ds(s * RPT, RPT)])
    plsc.subcore_barrier()

    def start_chunk(j, b):
        pltpu.async_copy(hs_hbm.at[src_v.at[j]], rows_v.at[b], gsems[b])
        pltpu.async_copy(dst_hbm.at[wid * ch + j], dstr_v.at[b], dsems[b])

    def issue_scatter(j, b):
        pltpu.make_async_copy(hs_hbm.at[src_v.at[j]], rows_v.at[b], gsems[b]).wait()
        pltpu.make_async_copy(dst_hbm.at[wid * ch + j], dstr_v.at[b], dsems[b]).wait()
        pltpu.async_copy(rows_v.at[b], acc.at[dstr_v.at[b]], ssems[b], add=True)

    def wait_scatter(b):
        pltpu.make_async_copy(rows_v.at[b], acc.at[dstr_v.at[b]], ssems[b]).wait()

    for b in range(NBUF):
        start_chunk(b, b)

    ngroups = ch // NBUF - 1

    def group(g, carry):
        j0 = g * NBUF
        for b in range(NBUF):
            issue_scatter(j0 + b, b)
        for b in range(NBUF):
            wait_scatter(b)
            start_chunk(j0 + NBUF + b, b)
        return carry

    lax.fori_loop(0, ngroups, group, 0)

    j0 = ngroups * NBUF
    for b in range(NBUF):
        issue_scatter(j0 + b, b)
    for b in range(NBUF):
        wait_scatter(b)

    plsc.subcore_barrier()
    pltpu.sync_copy(acc.at[pl.ds(s * RPT, RPT)], out_hbm.at[c, pl.ds(s * RPT, RPT)])


def _make_deg(ch):
    return pl.kernel(
        functools.partial(_deg_body, ch=ch),
        out_type=jax.ShapeDtypeStruct((NC, N_ACC, DEGW), jnp.float32),
        mesh=_sc_mesh,
        scratch_types=[
            pltpu.VMEM((ch, LANE), jnp.int32),
            pltpu.VMEM((LANE, DEGW), jnp.float32),
            pltpu.SemaphoreType.DMA,
            pltpu.VMEM_SHARED((N_ACC, DEGW), jnp.float32),
        ],
    )


def _make_agg(ch):
    return pl.kernel(
        functools.partial(_agg_body, ch=ch),
        out_type=jax.ShapeDtypeStruct((NC, N_ACC, D), jnp.float32),
        mesh=_sc_mesh,
        scratch_types=[
            pltpu.VMEM((ch, LANE), jnp.int32),
            pltpu.VMEM((NBUF, LANE), jnp.int32),
            pltpu.VMEM((NBUF, LANE, D), jnp.float32),
            [pltpu.SemaphoreType.DMA] * NBUF,
            [pltpu.SemaphoreType.DMA] * NBUF,
            [pltpu.SemaphoreType.DMA] * NBUF,
            pltpu.VMEM_SHARED((N_ACC, D), jnp.float32),
        ],
    )


_RB = 400
_GRID = N // _RB


def _tc_first_body(deg_ref, x_ref, w_ref, dis_ref, hs_ref):
    deg = deg_ref[0, :, 0:1] + deg_ref[1, :, 0:1] + 1.0
    dis = lax.rsqrt(jnp.maximum(deg, 1.0))
    dis_ref[...] = dis
    hs_ref[...] = dis * jnp.dot(x_ref[...], w_ref[...], preferred_element_type=jnp.float32)


def _tc_mid_body(agg_ref, hs_ref, dis_ref, b_ref, w_ref, out_ref):
    dis = dis_ref[...]
    t = (agg_ref[0] + agg_ref[1] + hs_ref[...]) * dis + b_ref[...]
    h = jnp.maximum(t, 0.0)
    out_ref[...] = dis * jnp.dot(h, w_ref[...], preferred_element_type=jnp.float32)


def _tc_last_body(agg_ref, hs_ref, dis_ref, b_ref, out_ref):
    out_ref[...] = (agg_ref[0] + agg_ref[1] + hs_ref[...]) * dis_ref[...] + b_ref[...]


_tc_first = pl.pallas_call(
    _tc_first_body,
    grid=(_GRID,),
    in_specs=[
        pl.BlockSpec((NC, _RB, DEGW), lambda i: (0, i, 0)),
        pl.BlockSpec((_RB, D), lambda i: (i, 0)),
        pl.BlockSpec((D, D), lambda i: (0, 0)),
    ],
    out_specs=[
        pl.BlockSpec((_RB, 1), lambda i: (i, 0)),
        pl.BlockSpec((_RB, D), lambda i: (i, 0)),
    ],
    out_shape=[
        jax.ShapeDtypeStruct((N, 1), jnp.float32),
        jax.ShapeDtypeStruct((N, D), jnp.float32),
    ],
)

_tc_mid = pl.pallas_call(
    _tc_mid_body,
    grid=(_GRID,),
    in_specs=[
        pl.BlockSpec((NC, _RB, D), lambda i: (0, i, 0)),
        pl.BlockSpec((_RB, D), lambda i: (i, 0)),
        pl.BlockSpec((_RB, 1), lambda i: (i, 0)),
        pl.BlockSpec((1, D), lambda i: (0, 0)),
        pl.BlockSpec((D, D), lambda i: (0, 0)),
    ],
    out_specs=pl.BlockSpec((_RB, D), lambda i: (i, 0)),
    out_shape=jax.ShapeDtypeStruct((N, D), jnp.float32),
)

_tc_last = pl.pallas_call(
    _tc_last_body,
    grid=(_GRID,),
    in_specs=[
        pl.BlockSpec((NC, _RB, D), lambda i: (0, i, 0)),
        pl.BlockSpec((_RB, D), lambda i: (i, 0)),
        pl.BlockSpec((_RB, 1), lambda i: (i, 0)),
        pl.BlockSpec((1, D), lambda i: (0, 0)),
    ],
    out_specs=pl.BlockSpec((_RB, D), lambda i: (i, 0)),
    out_shape=jax.ShapeDtypeStruct((N, D), jnp.float32),
)


def kernel(x, edge_index, W1, b1, W2, b2, W3, b3):
    n, d = x.shape
    e = edge_index.shape[1]
    ch = -(-e // (NT * LANE))
    ch = -(-ch // 8) * 8
    e_pad = NT * ch * LANE
    pad = e_pad - e

    src = jnp.concatenate([edge_index[0], jnp.zeros((pad,), jnp.int32)])
    dst = jnp.concatenate([edge_index[1], jnp.full((pad,), n, jnp.int32)])
    src2d = src.reshape(NT * ch, LANE)
    dst2d = dst.reshape(NT * ch, LANE)

    zrows = jnp.zeros((RPT, D), jnp.float32)
    ones = jnp.ones((LANE, DEGW), jnp.float32)

    deg_p = _make_deg(ch)(dst2d, zrows, ones)
    agg_fn = _make_agg(ch)

    dis, hs = _tc_first(deg_p, x, W1)

    agg = agg_fn(hs, src2d, dst2d, zrows)
    hs = _tc_mid(agg, hs, dis, b1.reshape(1, D), W2)

    agg = agg_fn(hs, src2d, dst2d, zrows)
    hs = _tc_mid(agg, hs, dis, b2.reshape(1, D), W3)

    agg = agg_fn(hs, src2d, dst2d, zrows)
    out = _tc_last(agg, hs, dis, b3.reshape(1, D))
    return out


# speedup vs baseline: 19.4445x; 2.4580x over previous
import functools

import jax
import jax.numpy as jnp
from jax import lax
from jax.experimental import pallas as pl
from jax.experimental.pallas import tpu as pltpu
from jax.experimental.pallas import tpu_sc as plsc

N = 10000
D = 128
LANE = 128
NC, NS = 2, 16
NT = NC * NS
N_ACC = 10240
RPT = N_ACC // NS
DEGW = 128

_sc_mesh = plsc.VectorSubcoreMesh(core_axis_name="c", subcore_axis_name="s")


def _deg_body(dst_hbm, zeros_hbm, ones_hbm, out_hbm, dst_v, ones_v, sem, acc, *, ch):
    c = lax.axis_index("c")
    s = lax.axis_index("s")
    wid = c * NS + s
    pltpu.sync_copy(dst_hbm.at[pl.ds(wid * ch, ch)], dst_v)
    pltpu.sync_copy(ones_hbm, ones_v)
    pltpu.sync_copy(zeros_hbm, acc.at[pl.ds(s * RPT, RPT)])
    plsc.subcore_barrier()

    def fire(j, carry):
        pltpu.async_copy(ones_v, acc.at[dst_v.at[j]], sem, add=True)
        return carry

    lax.fori_loop(0, ch, fire, 0)

    def drain(j, carry):
        pltpu.make_async_copy(ones_v, acc.at[dst_v.at[j]], sem).wait()
        return carry

    lax.fori_loop(0, ch, drain, 0)
    plsc.subcore_barrier()
    pltpu.sync_copy(acc.at[pl.ds(s * RPT, RPT)], out_hbm.at[c, pl.ds(s * RPT, RPT)])


NBUF = 2


def _agg_body(hs_hbm, src_hbm, dst_hbm, zeros_hbm, out_hbm, src_v, dstr_v,
              rows_v, gsems, dsems, ssems, acc, *, ch):
    c = lax.axis_index("c")
    s = lax.axis_index("s")
    wid = c * NS + s
    pltpu.sync_copy(src_hbm.at[pl.ds(wid * ch, ch)], src_v)
    pltpu.sync_copy(zeros_hbm, acc.at[pl.ds(s * RPT, RPT)])
    plsc.subcore_barrier()

    def start_chunk(j, b):
        pltpu.async_copy(hs_hbm.at[src_v.at[j]], rows_v.at[b], gsems[b])
        pltpu.async_copy(dst_hbm.at[wid * ch + j], dstr_v.at[b], dsems[b])

    def issue_scatter(j, b):
        pltpu.make_async_copy(hs_hbm.at[src_v.at[j]], rows_v.at[b], gsems[b]).wait()
        pltpu.make_async_copy(dst_hbm.at[wid * ch + j], dstr_v.at[b], dsems[b]).wait()
        pltpu.async_copy(rows_v.at[b], acc.at[dstr_v.at[b]], ssems[b], add=True)

    def wait_scatter(b):
        pltpu.make_async_copy(rows_v.at[b], acc.at[dstr_v.at[b]], ssems[b]).wait()

    for b in range(NBUF):
        start_chunk(b, b)

    ngroups = ch // NBUF - 1

    def group(g, carry):
        j0 = g * NBUF
        for b in range(NBUF):
            issue_scatter(j0 + b, b)
        for b in range(NBUF):
            wait_scatter(b)
            start_chunk(j0 + NBUF + b, b)
        return carry

    lax.fori_loop(0, ngroups, group, 0)

    j0 = ngroups * NBUF
    for b in range(NBUF):
        issue_scatter(j0 + b, b)
    for b in range(NBUF):
        wait_scatter(b)

    plsc.subcore_barrier()
    pltpu.sync_copy(acc.at[pl.ds(s * RPT, RPT)], out_hbm.at[c, pl.ds(s * RPT, RPT)])


def _make_deg(ch):
    return pl.kernel(
        functools.partial(_deg_body, ch=ch),
        out_type=jax.ShapeDtypeStruct((NC, N_ACC, DEGW), jnp.float32),
        mesh=_sc_mesh,
        scratch_types=[
            pltpu.VMEM((ch, LANE), jnp.int32),
            pltpu.VMEM((LANE, DEGW), jnp.float32),
            pltpu.SemaphoreType.DMA,
            pltpu.VMEM_SHARED((N_ACC, DEGW), jnp.float32),
        ],
    )


def _make_agg(ch):
    return pl.kernel(
        functools.partial(_agg_body, ch=ch),
        out_type=jax.ShapeDtypeStruct((NC, N_ACC, D), jnp.float32),
        mesh=_sc_mesh,
        scratch_types=[
            pltpu.VMEM((ch, LANE), jnp.int32),
            pltpu.VMEM((NBUF, LANE), jnp.int32),
            pltpu.VMEM((NBUF, LANE, D), jnp.float32),
            [pltpu.SemaphoreType.DMA] * NBUF,
            [pltpu.SemaphoreType.DMA] * NBUF,
            [pltpu.SemaphoreType.DMA] * NBUF,
            pltpu.VMEM_SHARED((N_ACC, D), jnp.float32),
        ],
    )


_RB = 400
_GRID = N // _RB


def _tc_first_body(deg_ref, x_ref, w_ref, dis_ref, hs_ref):
    deg = deg_ref[0, :, 0:1] + deg_ref[1, :, 0:1] + 1.0
    dis = lax.rsqrt(jnp.maximum(deg, 1.0))
    dis_ref[...] = dis
    hs_ref[...] = dis * jnp.dot(x_ref[...], w_ref[...], preferred_element_type=jnp.float32)


def _tc_mid_body(agg_ref, hs_ref, dis_ref, b_ref, w_ref, out_ref):
    dis = dis_ref[...]
    t = (agg_ref[0] + agg_ref[1] + hs_ref[...]) * dis + b_ref[...]
    h = jnp.maximum(t, 0.0)
    out_ref[...] = dis * jnp.dot(h, w_ref[...], preferred_element_type=jnp.float32)


def _tc_last_body(agg_ref, hs_ref, dis_ref, b_ref, out_ref):
    out_ref[...] = (agg_ref[0] + agg_ref[1] + hs_ref[...]) * dis_ref[...] + b_ref[...]


_tc_first = pl.pallas_call(
    _tc_first_body,
    grid=(_GRID,),
    in_specs=[
        pl.BlockSpec((NC, _RB, DEGW), lambda i: (0, i, 0)),
        pl.BlockSpec((_RB, D), lambda i: (i, 0)),
        pl.BlockSpec((D, D), lambda i: (0, 0)),
    ],
    out_specs=[
        pl.BlockSpec((_RB, 1), lambda i: (i, 0)),
        pl.BlockSpec((_RB, D), lambda i: (i, 0)),
    ],
    out_shape=[
        jax.ShapeDtypeStruct((N, 1), jnp.float32),
        jax.ShapeDtypeStruct((N, D), jnp.float32),
    ],
)

_tc_mid = pl.pallas_call(
    _tc_mid_body,
    grid=(_GRID,),
    in_specs=[
        pl.BlockSpec((NC, _RB, D), lambda i: (0, i, 0)),
        pl.BlockSpec((_RB, D), lambda i: (i, 0)),
        pl.BlockSpec((_RB, 1), lambda i: (i, 0)),
        pl.BlockSpec((1, D), lambda i: (0, 0)),
        pl.BlockSpec((D, D), lambda i: (0, 0)),
    ],
    out_specs=pl.BlockSpec((_RB, D), lambda i: (i, 0)),
    out_shape=jax.ShapeDtypeStruct((N, D), jnp.float32),
)

_tc_last = pl.pallas_call(
    _tc_last_body,
    grid=(_GRID,),
    in_specs=[
        pl.BlockSpec((NC, _RB, D), lambda i: (0, i, 0)),
        pl.BlockSpec((_RB, D), lambda i: (i, 0)),
        pl.BlockSpec((_RB, 1), lambda i: (i, 0)),
        pl.BlockSpec((1, D), lambda i: (0, 0)),
    ],
    out_specs=pl.BlockSpec((_RB, D), lambda i: (i, 0)),
    out_shape=jax.ShapeDtypeStruct((N, D), jnp.float32),
)


def kernel(x, edge_index, W1, b1, W2, b2, W3, b3):
    n, d = x.shape
    e = edge_index.shape[1]
    ch = -(-e // (NT * LANE))
    ch = -(-ch // 8) * 8
    e_pad = NT * ch * LANE
    pad = e_pad - e

    ar = jnp.arange(pad, dtype=jnp.int32)
    src = jnp.concatenate([edge_index[0], ar % n])
    dst = jnp.concatenate([edge_index[1], n + ar % (N_ACC - n)])
    src2d = src.reshape(NT * ch, LANE)
    dst2d = dst.reshape(NT * ch, LANE)

    zrows = jnp.zeros((RPT, D), jnp.float32)
    ones = jnp.ones((LANE, DEGW), jnp.float32)

    deg_p = _make_deg(ch)(dst2d, zrows, ones)
    agg_fn = _make_agg(ch)

    dis, hs = _tc_first(deg_p, x, W1)

    agg = agg_fn(hs, src2d, dst2d, zrows)
    hs = _tc_mid(agg, hs, dis, b1.reshape(1, D), W2)

    agg = agg_fn(hs, src2d, dst2d, zrows)
    hs = _tc_mid(agg, hs, dis, b2.reshape(1, D), W3)

    agg = agg_fn(hs, src2d, dst2d, zrows)
    out = _tc_last(agg, hs, dis, b3.reshape(1, D))
    return out


# speedup vs baseline: 21.7731x; 1.1198x over previous
import functools

import jax
import jax.numpy as jnp
from jax import lax
from jax.experimental import pallas as pl
from jax.experimental.pallas import tpu as pltpu
from jax.experimental.pallas import tpu_sc as plsc

N = 10000
D = 128
LANE = 128
NC, NS = 2, 16
NT = NC * NS
N_ACC = 10112
RPT = N_ACC // NS
DEGW = 128

_sc_mesh = plsc.VectorSubcoreMesh(core_axis_name="c", subcore_axis_name="s")


def _deg_body(dst_hbm, zeros_hbm, ones_hbm, out_hbm, dst_v, ones_v, sem, acc, *, ch):
    c = lax.axis_index("c")
    s = lax.axis_index("s")
    wid = c * NS + s
    pltpu.sync_copy(dst_hbm.at[pl.ds(wid * ch, ch)], dst_v)
    pltpu.sync_copy(ones_hbm, ones_v)
    pltpu.sync_copy(zeros_hbm, acc.at[pl.ds(s * RPT, RPT)])
    plsc.subcore_barrier()

    def fire(j, carry):
        pltpu.async_copy(ones_v, acc.at[dst_v.at[j, 0]], sem, add=True)
        return carry

    lax.fori_loop(0, ch, fire, 0)

    def drain(j, carry):
        pltpu.make_async_copy(ones_v, acc.at[dst_v.at[j, 0]], sem).wait()
        return carry

    lax.fori_loop(0, ch, drain, 0)
    plsc.subcore_barrier()
    pltpu.sync_copy(acc.at[pl.ds(s * RPT, RPT)], out_hbm.at[c, pl.ds(s * RPT, RPT)])


NBUF = 3


def _agg_body(hs_hbm, src_hbm, dst_hbm, zeros_hbm, out_hbm, sid_v, did_v,
              rows_v, isems, dsems, gsems, ssems, acc, *, ch):
    c = lax.axis_index("c")
    s = lax.axis_index("s")
    wid = c * NS + s
    base = wid * ch
    pltpu.sync_copy(zeros_hbm, acc.at[pl.ds(s * RPT, RPT)])
    plsc.subcore_barrier()

    def load_src(j, b):
        pltpu.async_copy(src_hbm.at[base + j], sid_v.at[b], isems[b])

    def wait_src(j, b):
        pltpu.make_async_copy(src_hbm.at[base + j], sid_v.at[b], isems[b]).wait()

    def load_dst(j, b):
        pltpu.async_copy(dst_hbm.at[base + j], did_v.at[b], dsems[b])

    def wait_dst(j, b):
        pltpu.make_async_copy(dst_hbm.at[base + j], did_v.at[b], dsems[b]).wait()

    def start_gather(j, b):
        pltpu.async_copy(hs_hbm.at[sid_v.at[b, 0]], rows_v.at[b], gsems[b])

    def wait_gather(j, b):
        pltpu.make_async_copy(hs_hbm.at[sid_v.at[b, 0]], rows_v.at[b], gsems[b]).wait()

    def issue_scatter(j, b):
        pltpu.async_copy(rows_v.at[b], acc.at[did_v.at[b, 0]], ssems[b], add=True)

    def wait_scatter(b):
        pltpu.make_async_copy(rows_v.at[b], acc.at[did_v.at[b, 0]], ssems[b]).wait()

    for b in range(NBUF):
        load_src(b, b)
        load_dst(b, b)
    for b in range(NBUF):
        wait_src(b, b)
        start_gather(b, b)

    ngroups = ch // NBUF - 1

    def group(g, carry):
        j0 = g * NBUF
        for b in range(NBUF):
            wait_gather(j0 + b, b)
            load_src(j0 + NBUF + b, b)
            wait_dst(j0 + b, b)
            issue_scatter(j0 + b, b)
        for b in range(NBUF):
            wait_scatter(b)
            load_dst(j0 + NBUF + b, b)
            wait_src(j0 + NBUF + b, b)
            start_gather(j0 + NBUF + b, b)
        return carry

    lax.fori_loop(0, ngroups, group, 0)

    j0 = ngroups * NBUF
    for b in range(NBUF):
        wait_gather(j0 + b, b)
        wait_dst(j0 + b, b)
        issue_scatter(j0 + b, b)
    for b in range(NBUF):
        wait_scatter(b)

    plsc.subcore_barrier()
    pltpu.sync_copy(acc.at[pl.ds(s * RPT, RPT)], out_hbm.at[c, pl.ds(s * RPT, RPT)])


def _make_deg(ch):
    return pl.kernel(
        functools.partial(_deg_body, ch=ch),
        out_type=jax.ShapeDtypeStruct((NC, N_ACC, DEGW), jnp.float32),
        mesh=_sc_mesh,
        scratch_types=[
            pltpu.VMEM((ch, 1, LANE), jnp.int32),
            pltpu.VMEM((LANE, DEGW), jnp.float32),
            pltpu.SemaphoreType.DMA,
            pltpu.VMEM_SHARED((N_ACC, DEGW), jnp.float32),
        ],
    )


def _make_agg(ch):
    return pl.kernel(
        functools.partial(_agg_body, ch=ch),
        out_type=jax.ShapeDtypeStruct((NC, N_ACC, D), jnp.float32),
        mesh=_sc_mesh,
        scratch_types=[
            pltpu.VMEM((NBUF, 1, LANE), jnp.int32),
            pltpu.VMEM((NBUF, 1, LANE), jnp.int32),
            pltpu.VMEM((NBUF, LANE, D), jnp.float32),
            [pltpu.SemaphoreType.DMA] * NBUF,
            [pltpu.SemaphoreType.DMA] * NBUF,
            [pltpu.SemaphoreType.DMA] * NBUF,
            [pltpu.SemaphoreType.DMA] * NBUF,
            pltpu.VMEM_SHARED((N_ACC, D), jnp.float32),
        ],
    )


_RB = 400
_GRID = N // _RB


def _tc_first_body(deg_ref, x_ref, w_ref, dis_ref, hs_ref):
    deg = deg_ref[0, :, 0:1] + deg_ref[1, :, 0:1] + 1.0
    dis = lax.rsqrt(jnp.maximum(deg, 1.0))
    dis_ref[...] = dis
    hs_ref[...] = dis * jnp.dot(x_ref[...], w_ref[...], preferred_element_type=jnp.float32)


def _tc_mid_body(agg_ref, hs_ref, dis_ref, b_ref, w_ref, out_ref):
    dis = dis_ref[...]
    t = (agg_ref[0] + agg_ref[1] + hs_ref[...]) * dis + b_ref[...]
    h = jnp.maximum(t, 0.0)
    out_ref[...] = dis * jnp.dot(h, w_ref[...], preferred_element_type=jnp.float32)


def _tc_last_body(agg_ref, hs_ref, dis_ref, b_ref, out_ref):
    out_ref[...] = (agg_ref[0] + agg_ref[1] + hs_ref[...]) * dis_ref[...] + b_ref[...]


_tc_first = pl.pallas_call(
    _tc_first_body,
    grid=(_GRID,),
    in_specs=[
        pl.BlockSpec((NC, _RB, DEGW), lambda i: (0, i, 0)),
        pl.BlockSpec((_RB, D), lambda i: (i, 0)),
        pl.BlockSpec((D, D), lambda i: (0, 0)),
    ],
    out_specs=[
        pl.BlockSpec((_RB, 1), lambda i: (i, 0)),
        pl.BlockSpec((_RB, D), lambda i: (i, 0)),
    ],
    out_shape=[
        jax.ShapeDtypeStruct((N, 1), jnp.float32),
        jax.ShapeDtypeStruct((N, D), jnp.float32),
    ],
)

_tc_mid = pl.pallas_call(
    _tc_mid_body,
    grid=(_GRID,),
    in_specs=[
        pl.BlockSpec((NC, _RB, D), lambda i: (0, i, 0)),
        pl.BlockSpec((_RB, D), lambda i: (i, 0)),
        pl.BlockSpec((_RB, 1), lambda i: (i, 0)),
        pl.BlockSpec((1, D), lambda i: (0, 0)),
        pl.BlockSpec((D, D), lambda i: (0, 0)),
    ],
    out_specs=pl.BlockSpec((_RB, D), lambda i: (i, 0)),
    out_shape=jax.ShapeDtypeStruct((N, D), jnp.float32),
)

_tc_last = pl.pallas_call(
    _tc_last_body,
    grid=(_GRID,),
    in_specs=[
        pl.BlockSpec((NC, _RB, D), lambda i: (0, i, 0)),
        pl.BlockSpec((_RB, D), lambda i: (i, 0)),
        pl.BlockSpec((_RB, 1), lambda i: (i, 0)),
        pl.BlockSpec((1, D), lambda i: (0, 0)),
    ],
    out_specs=pl.BlockSpec((_RB, D), lambda i: (i, 0)),
    out_shape=jax.ShapeDtypeStruct((N, D), jnp.float32),
)


def kernel(x, edge_index, W1, b1, W2, b2, W3, b3):
    n, d = x.shape
    e = edge_index.shape[1]
    ch = -(-e // (NT * LANE))
    ch = -(-ch // NBUF) * NBUF
    e_pad = NT * ch * LANE
    pad = e_pad - e

    ar = jnp.arange(pad, dtype=jnp.int32)
    src = jnp.concatenate([edge_index[0], ar % n])
    dst = jnp.concatenate([edge_index[1], n + ar % (N_ACC - n)])
    src2d = src.reshape(NT * ch, 1, LANE)
    dst2d = dst.reshape(NT * ch, 1, LANE)

    zrows = jnp.zeros((RPT, D), jnp.float32)
    ones = jnp.ones((LANE, DEGW), jnp.float32)

    deg_p = _make_deg(ch)(dst2d, zrows, ones)
    agg_fn = _make_agg(ch)

    dis, hs = _tc_first(deg_p, x, W1)

    agg = agg_fn(hs, src2d, dst2d, zrows)
    hs = _tc_mid(agg, hs, dis, b1.reshape(1, D), W2)

    agg = agg_fn(hs, src2d, dst2d, zrows)
    hs = _tc_mid(agg, hs, dis, b2.reshape(1, D), W3)

    agg = agg_fn(hs, src2d, dst2d, zrows)
    out = _tc_last(agg, hs, dis, b3.reshape(1, D))
    return out
